# baseline (speedup 1.0000x reference)
import functools

import jax
import jax.numpy as jnp
import numpy as np
from jax import lax
from jax.experimental import pallas as pl
from jax.experimental.pallas import tpu as pltpu
from jax.experimental.pallas import tpu_sc as plsc

N_NODES = 10000
N_EDGES = 320000
HID = 128
HEADS = 4
C = HID // HEADS
INV_SQRT_C = np.float32(1.0 / np.sqrt(C))

NB = 400
EB = 2000
CB = 64
NTILES = 32
NCHUNKS = N_EDGES // CB
NHALF = 5056
ACCR = 5120
NPT = ACCR // 16
N_PAD = 2 * ACCR


def _ln(x, g, b, eps=1e-5):
    m = jnp.mean(x, axis=-1, keepdims=True)
    v = jnp.mean((x - m) ** 2, axis=-1, keepdims=True)
    return (x - m) / jnp.sqrt(v + eps) * g + b


def _full(shape):
    return pl.BlockSpec(shape, lambda *_: tuple(0 for _ in shape))


def _f16(v):
    return jnp.full((16,), v, jnp.int32)


def _lane_perm(t, pm):
    return lax.gather(
        t, pm[:, None],
        dimension_numbers=lax.GatherDimensionNumbers(
            offset_dims=(), collapsed_slice_dims=(0,), start_index_map=(0,)),
        slice_sizes=(1,), mode=lax.GatherScatterMode.PROMISE_IN_BOUNDS)


_MESH = plsc.VectorSubcoreMesh(core_axis_name="c", subcore_axis_name="s",
                               num_cores=2, num_subcores=16)


def _relpos_body(xyzp_hbm, src_hbm, dst_hbm, rel_hbm,
                 idx_s, idx_d, xs, xd, rel, sem):
    w = lax.axis_index("c") * 16 + lax.axis_index("s")
    nchunks = (NCHUNKS // NTILES) + jnp.where(w < (NCHUNKS % NTILES), 1, 0)

    def chunk(j, _):
        base = pl.multiple_of((w + j * NTILES) * CB, CB)
        pltpu.sync_copy(src_hbm.at[pl.ds(base, CB)], idx_s)
        pltpu.sync_copy(dst_hbm.at[pl.ds(base, CB)], idx_d)
        pltpu.async_copy(xyzp_hbm.at[idx_s], xs, sem).wait()
        pltpu.async_copy(xyzp_hbm.at[idx_d], xd, sem).wait()

        def edge(i, _):
            rel[i, pl.ds(0, 16)] = xd[i, pl.ds(0, 16)] - xs[i, pl.ds(0, 16)]
            return 0

        lax.fori_loop(0, CB, edge, 0)
        pltpu.sync_copy(rel, rel_hbm.at[pl.ds(base, CB)])
        return 0

    lax.fori_loop(0, nchunks, chunk, 0)


def _relpos(xyzp, src, dst):
    kern = pl.kernel(
        _relpos_body,
        out_type=jax.ShapeDtypeStruct((N_EDGES, 16), jnp.float32),
        mesh=_MESH,
        scratch_types=[
            pltpu.VMEM((CB,), jnp.int32),
            pltpu.VMEM((CB,), jnp.int32),
            pltpu.VMEM((CB, HID), jnp.float32),
            pltpu.VMEM((CB, HID), jnp.float32),
            pltpu.VMEM((CB, 16), jnp.float32),
            pltpu.SemaphoreType.DMA,
        ],
    )
    return kern(xyzp, src, dst)


def _edge_body(src_hbm, dst_hbm, e_hbm, qh_hbm, kvh_hbm, out_hbm,
               idx_sa, idx_da, idx_dl, ebuf, qbuf, kvbuf, msg,
               idx_sa2, idx_da2, idx_dl2, ebuf2, qbuf2,
               kvbuf2, msg2, cmp, acc, sem, sem2):
    cix = lax.axis_index("c")
    sid = lax.axis_index("s")
    zero16 = jnp.zeros((16,), jnp.float32)
    iota16 = lax.iota(jnp.int32, 16)
    oh0 = jnp.where(iota16 == cix * 2, 1.0, 0.0).astype(jnp.float32)
    oh1 = jnp.where(iota16 == cix * 2 + 1, 1.0, 0.0).astype(jnp.float32)
    onehot = [oh0, oh1]
    perms = [jnp.bitwise_xor(iota16, k) for k in (1, 2, 4, 8)]
    nchunks = (NCHUNKS // 16) + jnp.where(sid < (NCHUNKS % 16), 1, 0)
    node_off = cix * N_NODES
    edge_off = cix * N_EDGES
    srow = pl.multiple_of(sid * NPT, 8)
    hA = jnp.full((16,), cix * 2, jnp.int32)
    hB = jnp.full((16,), cix * 2 + 1, jnp.int32)
    half = jnp.full((16,), 0x8000, jnp.uint32)
    topm = jnp.full((16,), 0xFFFF0000, jnp.uint32)

    for pss in range(2):
        pbase = pss * NHALF

        def zrow(r, _):
            for c9 in range(8):
                msg[r, pl.ds(c9 * 16, 16)] = zero16
            return 0

        lax.fori_loop(0, CB, zrow, 0)

        for t in range(NPT // CB):
            pltpu.sync_copy(msg, acc.at[pl.ds(srow + t * CB, CB)])
        plsc.subcore_barrier()

        def do_chunk(ci, idx_dlX, ebufX, qbufX, kvbufX, msgX):
            def grp(g, _):
                gb = g * 16
                for e in range(16):
                    i = gb + e
                    denv = zero16
                    for hl in range(2):
                        off = hl * C
                        q0 = qbufX[i, pl.ds(off, 16)]
                        q1 = qbufX[i, pl.ds(off + 16, 16)]
                        k0 = kvbufX[i, pl.ds(off, 16)]
                        k1 = kvbufX[i, pl.ds(off + 16, 16)]
                        e0 = ebufX[i, pl.ds(off, 16)]
                        e1 = ebufX[i, pl.ds(off + 16, 16)]
                        t = q0 * (k0 + e0) + q1 * (k1 + e1)
                        for pm in perms:
                            t = t + _lane_perm(t, pm)
                        exv = jnp.exp(t * INV_SQRT_C)
                        v0 = kvbufX[i, pl.ds(64 + off, 16)]
                        v1 = kvbufX[i, pl.ds(64 + off + 16, 16)]
                        msgX[i, pl.ds(off, 16)] = (v0 + e0) * exv
                        msgX[i, pl.ds(off + 16, 16)] = (v1 + e1) * exv
                        denv = denv + exv * onehot[hl]
                    msgX[i, pl.ds(64, 16)] = denv
                return 0

            lax.fori_loop(0, CB // 16, grp, 0)
            pltpu.sync_copy(msgX, acc.at[idx_dlX], add=True)

        def load_idx(ci, idx_saX, idx_daX, idx_dlX):
            base = pl.multiple_of(ci * CB, CB)
            pltpu.sync_copy(src_hbm.at[pl.ds(base, CB)], idx_saX)
            pltpu.sync_copy(dst_hbm.at[pl.ds(base, CB)], idx_daX)
            for g in range(CB // 16):
                sl = pl.ds(g * 16, 16)
                dl = idx_daX[sl] - pbase
                ok = (dl >= 0) & (dl < NHALF)
                idx_dlX[sl] = jnp.where(ok, dl, NHALF)
                idx_saX[sl] = idx_saX[sl] + node_off
                idx_daX[sl] = idx_daX[sl] + node_off

        def pair(j2, _):
            c0 = sid + (2 * j2) * 16
            c1 = sid + (2 * j2 + 1) * 16
            valid1 = (2 * j2 + 1) < nchunks
            load_idx(c0, idx_sa, idx_da, idx_dl)
            cpA_kv = pltpu.async_copy(kvh_hbm.at[idx_sa], kvbuf, sem)
            cpA_q = pltpu.async_copy(qh_hbm.at[idx_da], qbuf, sem)
            pltpu.sync_copy(e_hbm.at[pl.ds(edge_off +
                                           pl.multiple_of(c0 * CB, CB), CB)],
                            ebuf)
            descB_kv = pltpu.make_async_copy(kvh_hbm.at[idx_sa2], kvbuf2, sem2)
            descB_q = pltpu.make_async_copy(qh_hbm.at[idx_da2], qbuf2, sem2)

            @pl.when(valid1)
            def _():
                load_idx(c1, idx_sa2, idx_da2, idx_dl2)
                descB_kv.start()
                descB_q.start()
                pltpu.sync_copy(e_hbm.at[pl.ds(edge_off +
                                               pl.multiple_of(c1 * CB, CB),
                                               CB)], ebuf2)

            cpA_kv.wait()
            cpA_q.wait()
            do_chunk(c0, idx_dl, ebuf, qbuf, kvbuf, msg)

            @pl.when(valid1)
            def _():
                descB_kv.wait()
                descB_q.wait()
                do_chunk(c1, idx_dl2, ebuf2, qbuf2, kvbuf2, msg2)

            return 0

        lax.fori_loop(0, (nchunks + 1) // 2, pair, 0)
        plsc.subcore_barrier()

        def wout(j, _):
            r0 = pl.multiple_of(srow + j * 8, 8)
            pltpu.sync_copy(acc.at[pl.ds(r0, 8)], qbuf.at[pl.ds(0, 8)])
            for r in range(8):
                dv = qbuf[r, pl.ds(64, 16)]
                ra = 1.0 / (_lane_perm(dv, hA) + 1e-16)
                rb = 1.0 / (_lane_perm(dv, hB) + 1e-16)
                for g, rr in ((0, ra), (1, rb)):
                    m0 = qbuf[r, pl.ds(g * 32, 16)] * rr
                    m1 = qbuf[r, pl.ds(g * 32 + 16, 16)] * rr
                    i0 = lax.bitcast_convert_type(m0, jnp.uint32) + half
                    i1 = lax.bitcast_convert_type(m1, jnp.uint32) + half
                    word = lax.shift_right_logical(i0, jnp.uint32(16)) | (i1 & topm)
                    cmp[r, pl.ds(g * 16, 16)] = word
            pltpu.sync_copy(cmp, out_hbm.at[cix, pl.ds(pss * ACCR + r0, 8)])
            return 0

        lax.fori_loop(0, NPT // 8, wout, 0)
        plsc.subcore_barrier()


def _edge_phase_sc(src, dst, e_l, qh, kvh):
    kern = pl.kernel(
        _edge_body,
        out_type=jax.ShapeDtypeStruct((2, N_PAD, 32), jnp.uint32),
        mesh=_MESH,
        scratch_types=[
            pltpu.VMEM((CB,), jnp.int32),
            pltpu.VMEM((CB,), jnp.int32),
            pltpu.VMEM((CB,), jnp.int32),
            pltpu.VMEM((CB, 64), jnp.float32),
            pltpu.VMEM((CB, HID), jnp.float32),
            pltpu.VMEM((CB, HID), jnp.float32),
            pltpu.VMEM((CB, HID), jnp.float32),
            pltpu.VMEM((CB,), jnp.int32),
            pltpu.VMEM((CB,), jnp.int32),
            pltpu.VMEM((CB,), jnp.int32),
            pltpu.VMEM((CB, 64), jnp.float32),
            pltpu.VMEM((CB, HID), jnp.float32),
            pltpu.VMEM((CB, HID), jnp.float32),
            pltpu.VMEM((CB, HID), jnp.float32),
            pltpu.VMEM((8, 32), jnp.uint32),
            pltpu.VMEM_SHARED((ACCR + 8, HID), jnp.float32),
            pltpu.SemaphoreType.DMA,
            pltpu.SemaphoreType.DMA,
        ],
    )
    return kern(src, dst, e_l, qh, kvh)


def _encoder_body(xin_ref, w_ref, b_ref, g_ref, bb_ref, o_ref):
    x = xin_ref[...]
    y = jnp.dot(x, w_ref[...], preferred_element_type=jnp.float32) + b_ref[...]
    y = _ln(y, g_ref[...], bb_ref[...])
    o_ref[...] = jnp.maximum(y, 0.0)


def _encoder(xin, p):
    n = xin.shape[0]
    return pl.pallas_call(
        _encoder_body,
        grid=(n // NB,),
        in_specs=[
            pl.BlockSpec((NB, 4), lambda i: (i, 0)),
            _full((4, HID)), _full((HID,)), _full((HID,)), _full((HID,)),
        ],
        out_specs=pl.BlockSpec((NB, HID), lambda i: (i, 0)),
        out_shape=jax.ShapeDtypeStruct((n, HID), jnp.float32),
    )(xin, p["enc_lin"]["w"], p["enc_lin"]["b"], p["enc_ln"]["g"], p["enc_ln"]["b"])


def _edge_mlp_body(rp_ref, w1_ref, b1_ref, g_ref, bg_ref, w2_ref, b2_ref,
                   we_ref, be_ref, e1_ref, e2_ref, e3_ref):
    rp = rp_ref[...]
    w1 = w1_ref[...]
    h = (rp[:, 0:1] * w1[0:1, :] + rp[:, 1:2] * w1[1:2, :]
         + rp[:, 2:3] * w1[2:3, :] + b1_ref[...])
    h = jnp.maximum(_ln(h, g_ref[...], bg_ref[...]), 0.0)
    ea = jnp.dot(h, w2_ref[...], preferred_element_type=jnp.float32) + b2_ref[...]
    we = we_ref[...]
    be = be_ref[...]
    for li, er in enumerate((e1_ref, e2_ref, e3_ref)):
        el = jnp.dot(ea, we[li], preferred_element_type=jnp.float32) + be[li]
        er[0] = el[:, :64]
        er[1] = el[:, 64:]


def _edge_mlp(rel_pos, p):
    e = rel_pos.shape[0]
    we = jnp.stack([lp["e"]["w"] for lp in p["layers"]])
    be = jnp.stack([lp["e"]["b"] for lp in p["layers"]])
    return pl.pallas_call(
        _edge_mlp_body,
        grid=(e // EB,),
        in_specs=[
            pl.BlockSpec((EB, 16), lambda i: (i, 0)),
            _full((3, HID)), _full((HID,)), _full((HID,)), _full((HID,)),
            _full((HID, HID)), _full((HID,)),
            _full((3, HID, HID)), _full((3, HID)),
        ],
        out_specs=[pl.BlockSpec((2, EB, 64), lambda i: (0, i, 0))] * 3,
        out_shape=[jax.ShapeDtypeStruct((2, e, 64), jnp.float32)] * 3,
    )(rel_pos, p["pos_lin1"]["w"], p["pos_lin1"]["b"], p["pos_ln"]["g"],
      p["pos_ln"]["b"], p["pos_lin2"]["w"], p["pos_lin2"]["b"], we, be)


def _qkv_body(x_ref, w_ref, b_ref, qh_ref, kvh_ref):
    x = x_ref[...]
    y = jnp.dot(x, w_ref[...], preferred_element_type=jnp.float32) + b_ref[...]
    z = jnp.zeros((NB, 64), jnp.float32)
    qh_ref[0] = jnp.concatenate([y[:, 0:64], z], axis=1)
    qh_ref[1] = jnp.concatenate([y[:, 64:128], z], axis=1)
    kvh_ref[0] = jnp.concatenate([y[:, 128:192], y[:, 256:320]], axis=1)
    kvh_ref[1] = jnp.concatenate([y[:, 192:256], y[:, 320:384]], axis=1)


def _qkv(x, lp):
    n = x.shape[0]
    w = jnp.concatenate([lp["q"]["w"], lp["k"]["w"], lp["v"]["w"]], axis=1)
    b = jnp.concatenate([lp["q"]["b"], lp["k"]["b"], lp["v"]["b"]])
    qh, kvh = pl.pallas_call(
        _qkv_body,
        grid=(n // NB,),
        in_specs=[pl.BlockSpec((NB, HID), lambda i: (i, 0)),
                  _full((HID, 3 * HID)), _full((3 * HID,))],
        out_specs=[pl.BlockSpec((2, NB, HID), lambda i: (0, i, 0)),
                   pl.BlockSpec((2, NB, HID), lambda i: (0, i, 0))],
        out_shape=[jax.ShapeDtypeStruct((2, n, HID), jnp.float32),
                   jax.ShapeDtypeStruct((2, n, HID), jnp.float32)],
    )(x, w, b)
    return qh.reshape(2 * n, HID), kvh.reshape(2 * n, HID)


def _post_body(p0_ref, p1_ref, x_ref, wskip_ref, bskip_ref, wb_ref,
               g1_ref, b1_ref, wf1_ref, bf1_ref, wf2_ref, bf2_ref,
               g2_ref, b2_ref, o_ref):
    topm = jnp.uint32(0xFFFF0000)

    def unpack(pu):
        cols = []
        for g in range(2):
            u = pu[:, g * 16:(g + 1) * 16]
            cols.append(lax.bitcast_convert_type(u << 16, jnp.float32))
            cols.append(lax.bitcast_convert_type(u & topm, jnp.float32))
        return jnp.concatenate(cols, axis=1)

    out = jnp.concatenate([unpack(p0_ref[...]), unpack(p1_ref[...])], axis=1)
    x = x_ref[...]
    xr = jnp.dot(x, wskip_ref[...], preferred_element_type=jnp.float32) + bskip_ref[...]
    wb = wb_ref[...]
    z = (jnp.sum(out * (wb[:HID, 0] + wb[2 * HID:, 0]), axis=-1, keepdims=True)
         + jnp.sum(xr * (wb[HID:2 * HID, 0] - wb[2 * HID:, 0]), axis=-1, keepdims=True))
    beta = 1.0 / (1.0 + jnp.exp(-z))
    h = beta * xr + (1.0 - beta) * out
    x1 = _ln(h + x, g1_ref[...], b1_ref[...])
    f = jnp.dot(x1, wf1_ref[...], preferred_element_type=jnp.float32) + bf1_ref[...]
    f = 0.5 * f * (1.0 + jax.lax.erf(f * np.float32(1.0 / np.sqrt(2.0))))
    f = jnp.dot(f, wf2_ref[...], preferred_element_type=jnp.float32) + bf2_ref[...]
    o_ref[...] = _ln(f + x1, g2_ref[...], b2_ref[...])


def _post(parts, x, lp):
    n = x.shape[0]
    return pl.pallas_call(
        _post_body,
        grid=(n // NB,),
        in_specs=[
            pl.BlockSpec((NB, 32), lambda i: (i, 0)),
            pl.BlockSpec((NB, 32), lambda i: (i, 0)),
            pl.BlockSpec((NB, HID), lambda i: (i, 0)),
            _full((HID, HID)), _full((HID,)), _full((3 * HID, 1)),
            _full((HID,)), _full((HID,)),
            _full((HID, 2 * HID)), _full((2 * HID,)),
            _full((2 * HID, HID)), _full((HID,)),
            _full((HID,)), _full((HID,)),
        ],
        out_specs=pl.BlockSpec((NB, HID), lambda i: (i, 0)),
        out_shape=jax.ShapeDtypeStruct((n, HID), jnp.float32),
    )(parts[0], parts[1], x, lp["skip"]["w"], lp["skip"]["b"], lp["beta"]["w"],
      lp["ln1"]["g"], lp["ln1"]["b"], lp["ffn1"]["w"], lp["ffn1"]["b"],
      lp["ffn2"]["w"], lp["ffn2"]["b"], lp["ln2"]["g"], lp["ln2"]["b"])


def _heads_body(x_ref, wc1_ref, bc1_ref, wc2_ref, bc2_ref,
                wb1_ref, bb1_ref, wb2_ref, bb2_ref, cls_ref, box_ref):
    x = x_ref[...]
    hc = jnp.maximum(jnp.dot(x, wc1_ref[...], preferred_element_type=jnp.float32)
                     + bc1_ref[...], 0.0)
    cls_ref[...] = jnp.dot(hc, wc2_ref[...], preferred_element_type=jnp.float32) + bc2_ref[...]
    hb = jnp.maximum(jnp.dot(x, wb1_ref[...], preferred_element_type=jnp.float32)
                     + bb1_ref[...], 0.0)
    box_ref[...] = jnp.dot(hb, wb2_ref[...], preferred_element_type=jnp.float32) + bb2_ref[...]


def _heads(x, p):
    n = x.shape[0]
    return pl.pallas_call(
        _heads_body,
        grid=(n // NB,),
        in_specs=[
            pl.BlockSpec((NB, HID), lambda i: (i, 0)),
            _full((HID, HID // 2)), _full((HID // 2,)),
            _full((HID // 2, 4)), _full((4,)),
            _full((HID, HID // 2)), _full((HID // 2,)),
            _full((HID // 2, 7)), _full((7,)),
        ],
        out_specs=[pl.BlockSpec((NB, 4), lambda i: (i, 0)),
                   pl.BlockSpec((NB, 7), lambda i: (i, 0))],
        out_shape=[jax.ShapeDtypeStruct((n, 4), jnp.float32),
                   jax.ShapeDtypeStruct((n, 7), jnp.float32)],
    )(x, p["cls1"]["w"], p["cls1"]["b"], p["cls2"]["w"], p["cls2"]["b"],
      p["box1"]["w"], p["box1"]["b"], p["box2"]["w"], p["box2"]["b"])


def kernel(xyz, attr, edge_index, params):
    src = edge_index[0]
    dst = edge_index[1]
    xin = jnp.concatenate([xyz, attr], axis=-1)
    x = _encoder(xin, params)
    xyzp = jnp.concatenate([xyz, jnp.zeros((xyz.shape[0], HID - 3), jnp.float32)], axis=1)
    rel_pos = _relpos(xyzp, src, dst)
    e123 = _edge_mlp(rel_pos, params)
    for li, lp in enumerate(params["layers"]):
        qh, kvh = _qkv(x, lp)
        pr = _edge_phase_sc(src, dst, e123[li].reshape(2 * N_EDGES, 64), qh, kvh)
        parts = jnp.concatenate(
            [pr[:, :NHALF], pr[:, ACCR:ACCR + (N_NODES - NHALF)]], axis=1)
        x = _post(parts, x, lp)
    return _heads(x, params)


# speedup vs baseline: 1.1351x; 1.1351x over previous
import functools

import jax
import jax.numpy as jnp
import numpy as np
from jax import lax
from jax.experimental import pallas as pl
from jax.experimental.pallas import tpu as pltpu
from jax.experimental.pallas import tpu_sc as plsc

N_NODES = 10000
N_EDGES = 320000
HID = 128
HEADS = 4
C = HID // HEADS
INV_SQRT_C = np.float32(1.0 / np.sqrt(C))

NB = 400
EB = 2000
CB = 128
NTILES = 32
NCHUNKS = N_EDGES // CB
NHALF = 5056
ACCR = 5120
NPT = ACCR // 16
N_PAD = 2 * ACCR


def _ln(x, g, b, eps=1e-5):
    m = jnp.mean(x, axis=-1, keepdims=True)
    v = jnp.mean((x - m) ** 2, axis=-1, keepdims=True)
    return (x - m) / jnp.sqrt(v + eps) * g + b


def _full(shape):
    return pl.BlockSpec(shape, lambda *_: tuple(0 for _ in shape))


def _f16(v):
    return jnp.full((16,), v, jnp.int32)


def _lane_perm(t, pm):
    return lax.gather(
        t, pm[:, None],
        dimension_numbers=lax.GatherDimensionNumbers(
            offset_dims=(), collapsed_slice_dims=(0,), start_index_map=(0,)),
        slice_sizes=(1,), mode=lax.GatherScatterMode.PROMISE_IN_BOUNDS)


_MESH = plsc.VectorSubcoreMesh(core_axis_name="c", subcore_axis_name="s",
                               num_cores=2, num_subcores=16)


def _relpos_body(xyzp_hbm, src_hbm, dst_hbm, rel_hbm,
                 idx_s, idx_d, xs, xd, rel, sem):
    w = lax.axis_index("c") * 16 + lax.axis_index("s")
    nchunks = (NCHUNKS // NTILES) + jnp.where(w < (NCHUNKS % NTILES), 1, 0)

    def chunk(j, _):
        base = pl.multiple_of((w + j * NTILES) * CB, CB)
        pltpu.sync_copy(src_hbm.at[pl.ds(base, CB)], idx_s)
        pltpu.sync_copy(dst_hbm.at[pl.ds(base, CB)], idx_d)
        pltpu.async_copy(xyzp_hbm.at[idx_s], xs, sem).wait()
        pltpu.async_copy(xyzp_hbm.at[idx_d], xd, sem).wait()

        def edge(i, _):
            rel[i, pl.ds(0, 16)] = xd[i, pl.ds(0, 16)] - xs[i, pl.ds(0, 16)]
            return 0

        lax.fori_loop(0, CB, edge, 0)
        pltpu.sync_copy(rel, rel_hbm.at[pl.ds(base, CB)])
        return 0

    lax.fori_loop(0, nchunks, chunk, 0)


def _relpos(xyzp, src, dst):
    kern = pl.kernel(
        _relpos_body,
        out_type=jax.ShapeDtypeStruct((N_EDGES, 16), jnp.float32),
        mesh=_MESH,
        scratch_types=[
            pltpu.VMEM((CB,), jnp.int32),
            pltpu.VMEM((CB,), jnp.int32),
            pltpu.VMEM((CB, HID), jnp.float32),
            pltpu.VMEM((CB, HID), jnp.float32),
            pltpu.VMEM((CB, 16), jnp.float32),
            pltpu.SemaphoreType.DMA,
        ],
    )
    return kern(xyzp, src, dst)


def _edge_body(src_hbm, dst_hbm, e_hbm, qh_hbm, kvh_hbm, out_hbm,
               idx_s, idx_d, idx_sa, idx_da, idx_dl, ebuf, qbuf, kvbuf, msg,
               stg, cmp, acc, sem):
    cix = lax.axis_index("c")
    sid = lax.axis_index("s")
    zero16 = jnp.zeros((16,), jnp.float32)
    iota16 = lax.iota(jnp.int32, 16)
    oh0 = jnp.where(iota16 == cix * 2, 1.0, 0.0).astype(jnp.float32)
    oh1 = jnp.where(iota16 == cix * 2 + 1, 1.0, 0.0).astype(jnp.float32)
    onehot = [oh0, oh1]
    perms = [jnp.bitwise_xor(iota16, k) for k in (1, 2, 4, 8)]
    nchunks = (NCHUNKS // 16) + jnp.where(sid < (NCHUNKS % 16), 1, 0)
    node_off = cix * N_NODES
    edge_off = cix * N_EDGES
    srow = pl.multiple_of(sid * NPT, 8)
    hA = jnp.full((16,), cix * 2, jnp.int32)
    hB = jnp.full((16,), cix * 2 + 1, jnp.int32)
    half = jnp.full((16,), 0x8000, jnp.uint32)
    topm = jnp.full((16,), 0xFFFF0000, jnp.uint32)

    for pss in range(2):
        pbase = pss * NHALF

        def zrow(r, _):
            for c9 in range(8):
                msg[r, pl.ds(c9 * 16, 16)] = zero16
            return 0

        lax.fori_loop(0, CB, zrow, 0)

        for t in range(NPT // CB):
            pltpu.sync_copy(msg, acc.at[pl.ds(srow + t * CB, CB)])
        pltpu.sync_copy(msg.at[pl.ds(0, NPT - (NPT // CB) * CB)],
                        acc.at[pl.ds(srow + (NPT // CB) * CB,
                                     NPT - (NPT // CB) * CB)])
        plsc.subcore_barrier()

        def chunk(j, _):
            base = pl.multiple_of((sid + j * 16) * CB, CB)
            pltpu.sync_copy(src_hbm.at[pl.ds(base, CB)], idx_s)
            pltpu.sync_copy(dst_hbm.at[pl.ds(base, CB)], idx_d)
            for g in range(CB // 16):
                sl = pl.ds(g * 16, 16)
                idx_sa[sl] = idx_s[sl] + node_off
                idx_da[sl] = idx_d[sl] + node_off
                dl = idx_d[sl] - pbase
                ok = (dl >= 0) & (dl < NHALF)
                idx_dl[sl] = jnp.where(ok, dl, NHALF)
            cp_kv = pltpu.async_copy(kvh_hbm.at[idx_sa], kvbuf, sem)
            cp_q = pltpu.async_copy(qh_hbm.at[idx_da], qbuf, sem)
            pltpu.sync_copy(e_hbm.at[pl.ds(edge_off + base, CB)], ebuf)
            cp_kv.wait()
            cp_q.wait()

            def grp(g, _):
                gb = g * 16
                for e in range(16):
                    i = gb + e
                    denv = zero16
                    for hl in range(2):
                        off = hl * C
                        q0 = qbuf[i, pl.ds(off, 16)]
                        q1 = qbuf[i, pl.ds(off + 16, 16)]
                        k0 = kvbuf[i, pl.ds(off, 16)]
                        k1 = kvbuf[i, pl.ds(off + 16, 16)]
                        e0 = ebuf[i, pl.ds(off, 16)]
                        e1 = ebuf[i, pl.ds(off + 16, 16)]
                        t = q0 * (k0 + e0) + q1 * (k1 + e1)
                        for pm in perms:
                            t = t + _lane_perm(t, pm)
                        exv = jnp.exp(t * INV_SQRT_C)
                        v0 = kvbuf[i, pl.ds(64 + off, 16)]
                        v1 = kvbuf[i, pl.ds(64 + off + 16, 16)]
                        msg[i, pl.ds(off, 16)] = (v0 + e0) * exv
                        msg[i, pl.ds(off + 16, 16)] = (v1 + e1) * exv
                        denv = denv + exv * onehot[hl]
                    msg[i, pl.ds(64, 16)] = denv
                return 0

            lax.fori_loop(0, CB // 16, grp, 0)
            pltpu.sync_copy(msg, acc.at[idx_dl], add=True)
            return 0

        lax.fori_loop(0, nchunks, chunk, 0)
        plsc.subcore_barrier()

        def wout(j, _):
            r0 = pl.multiple_of(srow + j * 8, 8)
            pltpu.sync_copy(acc.at[pl.ds(r0, 8)], stg)
            for r in range(8):
                dv = stg[r, pl.ds(64, 16)]
                ra = 1.0 / (_lane_perm(dv, hA) + 1e-16)
                rb = 1.0 / (_lane_perm(dv, hB) + 1e-16)
                for g, rr in ((0, ra), (1, rb)):
                    m0 = stg[r, pl.ds(g * 32, 16)] * rr
                    m1 = stg[r, pl.ds(g * 32 + 16, 16)] * rr
                    i0 = lax.bitcast_convert_type(m0, jnp.uint32) + half
                    i1 = lax.bitcast_convert_type(m1, jnp.uint32) + half
                    word = lax.shift_right_logical(i0, jnp.uint32(16)) | (i1 & topm)
                    cmp[r, pl.ds(g * 16, 16)] = word
            pltpu.sync_copy(cmp, out_hbm.at[cix, pl.ds(pss * ACCR + r0, 8)])
            return 0

        lax.fori_loop(0, NPT // 8, wout, 0)
        plsc.subcore_barrier()


def _edge_phase_sc(src, dst, e_l, qh, kvh):
    kern = pl.kernel(
        _edge_body,
        out_type=jax.ShapeDtypeStruct((2, N_PAD, 32), jnp.uint32),
        mesh=_MESH,
        scratch_types=[
            pltpu.VMEM((CB,), jnp.int32),
            pltpu.VMEM((CB,), jnp.int32),
            pltpu.VMEM((CB,), jnp.int32),
            pltpu.VMEM((CB,), jnp.int32),
            pltpu.VMEM((CB,), jnp.int32),
            pltpu.VMEM((CB, 64), jnp.float32),
            pltpu.VMEM((CB, HID), jnp.float32),
            pltpu.VMEM((CB, HID), jnp.float32),
            pltpu.VMEM((CB, HID), jnp.float32),
            pltpu.VMEM((8, HID), jnp.float32),
            pltpu.VMEM((8, 32), jnp.uint32),
            pltpu.VMEM_SHARED((ACCR + 8, HID), jnp.float32),
            pltpu.SemaphoreType.DMA,
        ],
    )
    return kern(src, dst, e_l, qh, kvh)


def _encoder_body(xin_ref, w_ref, b_ref, g_ref, bb_ref, o_ref):
    x = xin_ref[...]
    y = jnp.dot(x, w_ref[...], preferred_element_type=jnp.float32) + b_ref[...]
    y = _ln(y, g_ref[...], bb_ref[...])
    o_ref[...] = jnp.maximum(y, 0.0)


def _encoder(xin, p):
    n = xin.shape[0]
    return pl.pallas_call(
        _encoder_body,
        grid=(n // NB,),
        in_specs=[
            pl.BlockSpec((NB, 4), lambda i: (i, 0)),
            _full((4, HID)), _full((HID,)), _full((HID,)), _full((HID,)),
        ],
        out_specs=pl.BlockSpec((NB, HID), lambda i: (i, 0)),
        out_shape=jax.ShapeDtypeStruct((n, HID), jnp.float32),
    )(xin, p["enc_lin"]["w"], p["enc_lin"]["b"], p["enc_ln"]["g"], p["enc_ln"]["b"])


def _edge_mlp_body(rp_ref, w1_ref, b1_ref, g_ref, bg_ref, w2_ref, b2_ref,
                   we_ref, be_ref, e1_ref, e2_ref, e3_ref):
    rp = rp_ref[...]
    w1 = w1_ref[...]
    h = (rp[:, 0:1] * w1[0:1, :] + rp[:, 1:2] * w1[1:2, :]
         + rp[:, 2:3] * w1[2:3, :] + b1_ref[...])
    h = jnp.maximum(_ln(h, g_ref[...], bg_ref[...]), 0.0)
    ea = jnp.dot(h, w2_ref[...], preferred_element_type=jnp.float32) + b2_ref[...]
    we = we_ref[...]
    be = be_ref[...]
    for li, er in enumerate((e1_ref, e2_ref, e3_ref)):
        el = jnp.dot(ea, we[li], preferred_element_type=jnp.float32) + be[li]
        er[0] = el[:, :64]
        er[1] = el[:, 64:]


def _edge_mlp(rel_pos, p):
    e = rel_pos.shape[0]
    we = jnp.stack([lp["e"]["w"] for lp in p["layers"]])
    be = jnp.stack([lp["e"]["b"] for lp in p["layers"]])
    return pl.pallas_call(
        _edge_mlp_body,
        grid=(e // EB,),
        in_specs=[
            pl.BlockSpec((EB, 16), lambda i: (i, 0)),
            _full((3, HID)), _full((HID,)), _full((HID,)), _full((HID,)),
            _full((HID, HID)), _full((HID,)),
            _full((3, HID, HID)), _full((3, HID)),
        ],
        out_specs=[pl.BlockSpec((2, EB, 64), lambda i: (0, i, 0))] * 3,
        out_shape=[jax.ShapeDtypeStruct((2, e, 64), jnp.float32)] * 3,
    )(rel_pos, p["pos_lin1"]["w"], p["pos_lin1"]["b"], p["pos_ln"]["g"],
      p["pos_ln"]["b"], p["pos_lin2"]["w"], p["pos_lin2"]["b"], we, be)


def _qkv_body(x_ref, w_ref, b_ref, qh_ref, kvh_ref):
    x = x_ref[...]
    y = jnp.dot(x, w_ref[...], preferred_element_type=jnp.float32) + b_ref[...]
    z = jnp.zeros((NB, 64), jnp.float32)
    qh_ref[0] = jnp.concatenate([y[:, 0:64], z], axis=1)
    qh_ref[1] = jnp.concatenate([y[:, 64:128], z], axis=1)
    kvh_ref[0] = jnp.concatenate([y[:, 128:192], y[:, 256:320]], axis=1)
    kvh_ref[1] = jnp.concatenate([y[:, 192:256], y[:, 320:384]], axis=1)


def _qkv(x, lp):
    n = x.shape[0]
    w = jnp.concatenate([lp["q"]["w"], lp["k"]["w"], lp["v"]["w"]], axis=1)
    b = jnp.concatenate([lp["q"]["b"], lp["k"]["b"], lp["v"]["b"]])
    qh, kvh = pl.pallas_call(
        _qkv_body,
        grid=(n // NB,),
        in_specs=[pl.BlockSpec((NB, HID), lambda i: (i, 0)),
                  _full((HID, 3 * HID)), _full((3 * HID,))],
        out_specs=[pl.BlockSpec((2, NB, HID), lambda i: (0, i, 0)),
                   pl.BlockSpec((2, NB, HID), lambda i: (0, i, 0))],
        out_shape=[jax.ShapeDtypeStruct((2, n, HID), jnp.float32),
                   jax.ShapeDtypeStruct((2, n, HID), jnp.float32)],
    )(x, w, b)
    return qh.reshape(2 * n, HID), kvh.reshape(2 * n, HID)


def _post_body(p0_ref, p1_ref, x_ref, wskip_ref, bskip_ref, wb_ref,
               g1_ref, b1_ref, wf1_ref, bf1_ref, wf2_ref, bf2_ref,
               g2_ref, b2_ref, o_ref):
    topm = jnp.uint32(0xFFFF0000)

    def unpack(pu):
        cols = []
        for g in range(2):
            u = pu[:, g * 16:(g + 1) * 16]
            cols.append(lax.bitcast_convert_type(u << 16, jnp.float32))
            cols.append(lax.bitcast_convert_type(u & topm, jnp.float32))
        return jnp.concatenate(cols, axis=1)

    out = jnp.concatenate([unpack(p0_ref[...]), unpack(p1_ref[...])], axis=1)
    x = x_ref[...]
    xr = jnp.dot(x, wskip_ref[...], preferred_element_type=jnp.float32) + bskip_ref[...]
    wb = wb_ref[...]
    z = (jnp.sum(out * (wb[:HID, 0] + wb[2 * HID:, 0]), axis=-1, keepdims=True)
         + jnp.sum(xr * (wb[HID:2 * HID, 0] - wb[2 * HID:, 0]), axis=-1, keepdims=True))
    beta = 1.0 / (1.0 + jnp.exp(-z))
    h = beta * xr + (1.0 - beta) * out
    x1 = _ln(h + x, g1_ref[...], b1_ref[...])
    f = jnp.dot(x1, wf1_ref[...], preferred_element_type=jnp.float32) + bf1_ref[...]
    f = 0.5 * f * (1.0 + jax.lax.erf(f * np.float32(1.0 / np.sqrt(2.0))))
    f = jnp.dot(f, wf2_ref[...], preferred_element_type=jnp.float32) + bf2_ref[...]
    o_ref[...] = _ln(f + x1, g2_ref[...], b2_ref[...])


def _post(parts, x, lp):
    n = x.shape[0]
    return pl.pallas_call(
        _post_body,
        grid=(n // NB,),
        in_specs=[
            pl.BlockSpec((NB, 32), lambda i: (i, 0)),
            pl.BlockSpec((NB, 32), lambda i: (i, 0)),
            pl.BlockSpec((NB, HID), lambda i: (i, 0)),
            _full((HID, HID)), _full((HID,)), _full((3 * HID, 1)),
            _full((HID,)), _full((HID,)),
            _full((HID, 2 * HID)), _full((2 * HID,)),
            _full((2 * HID, HID)), _full((HID,)),
            _full((HID,)), _full((HID,)),
        ],
        out_specs=pl.BlockSpec((NB, HID), lambda i: (i, 0)),
        out_shape=jax.ShapeDtypeStruct((n, HID), jnp.float32),
    )(parts[0], parts[1], x, lp["skip"]["w"], lp["skip"]["b"], lp["beta"]["w"],
      lp["ln1"]["g"], lp["ln1"]["b"], lp["ffn1"]["w"], lp["ffn1"]["b"],
      lp["ffn2"]["w"], lp["ffn2"]["b"], lp["ln2"]["g"], lp["ln2"]["b"])


def _heads_body(x_ref, wc1_ref, bc1_ref, wc2_ref, bc2_ref,
                wb1_ref, bb1_ref, wb2_ref, bb2_ref, cls_ref, box_ref):
    x = x_ref[...]
    hc = jnp.maximum(jnp.dot(x, wc1_ref[...], preferred_element_type=jnp.float32)
                     + bc1_ref[...], 0.0)
    cls_ref[...] = jnp.dot(hc, wc2_ref[...], preferred_element_type=jnp.float32) + bc2_ref[...]
    hb = jnp.maximum(jnp.dot(x, wb1_ref[...], preferred_element_type=jnp.float32)
                     + bb1_ref[...], 0.0)
    box_ref[...] = jnp.dot(hb, wb2_ref[...], preferred_element_type=jnp.float32) + bb2_ref[...]


def _heads(x, p):
    n = x.shape[0]
    return pl.pallas_call(
        _heads_body,
        grid=(n // NB,),
        in_specs=[
            pl.BlockSpec((NB, HID), lambda i: (i, 0)),
            _full((HID, HID // 2)), _full((HID // 2,)),
            _full((HID // 2, 4)), _full((4,)),
            _full((HID, HID // 2)), _full((HID // 2,)),
            _full((HID // 2, 7)), _full((7,)),
        ],
        out_specs=[pl.BlockSpec((NB, 4), lambda i: (i, 0)),
                   pl.BlockSpec((NB, 7), lambda i: (i, 0))],
        out_shape=[jax.ShapeDtypeStruct((n, 4), jnp.float32),
                   jax.ShapeDtypeStruct((n, 7), jnp.float32)],
    )(x, p["cls1"]["w"], p["cls1"]["b"], p["cls2"]["w"], p["cls2"]["b"],
      p["box1"]["w"], p["box1"]["b"], p["box2"]["w"], p["box2"]["b"])


def kernel(xyz, attr, edge_index, params):
    src = edge_index[0]
    dst = edge_index[1]
    xin = jnp.concatenate([xyz, attr], axis=-1)
    x = _encoder(xin, params)
    xyzp = jnp.concatenate([xyz, jnp.zeros((xyz.shape[0], HID - 3), jnp.float32)], axis=1)
    rel_pos = _relpos(xyzp, src, dst)
    e123 = _edge_mlp(rel_pos, params)
    for li, lp in enumerate(params["layers"]):
        qh, kvh = _qkv(x, lp)
        pr = _edge_phase_sc(src, dst, e123[li].reshape(2 * N_EDGES, 64), qh, kvh)
        parts = jnp.concatenate(
            [pr[:, :NHALF], pr[:, ACCR:ACCR + (N_NODES - NHALF)]], axis=1)
        x = _post(parts, x, lp)
    return _heads(x, params)
